# baseline (speedup 1.0000x reference)
import functools

import jax
import jax.numpy as jnp
from jax import lax
from jax.experimental import pallas as pl
from jax.experimental.pallas import tpu as pltpu
from jax.experimental.pallas import tpu_sc as plsc

_NC = 2
_NS = 16
_NW = _NC * _NS

_DIM = 32
_B = 16384
_S = 200
_TOK = 512
_BT = _TOK // 128


@jax.jit
def _embed_gather_t(xt_flat, table):
    mesh = plsc.VectorSubcoreMesh(core_axis_name="c", subcore_axis_name="s")

    @functools.partial(
        pl.kernel,
        mesh=mesh,
        out_type=jax.ShapeDtypeStruct(
            (_S, _DIM // 8, _B // 128, 8, 128), jnp.float32
        ),
        scratch_types=[
            pltpu.VMEM((2, _TOK), jnp.int32),
            pltpu.VMEM((2, _TOK, _DIM), jnp.float32),
            pltpu.VMEM((2, _BT, _BT, 8, 128), jnp.float32),
            pltpu.SemaphoreType.DMA((2,)),
            pltpu.SemaphoreType.DMA((2,)),
        ],
        compiler_params=pltpu.CompilerParams(
            use_tc_tiling_on_sc=False, needs_layout_passes=False
        ),
    )
    def k(xt_hbm, table_hbm, out_hbm, idx_v, rows_v, out_v, gsem, osem):
        wid = lax.axis_index("s") * _NC + lax.axis_index("c")
        b0 = wid * _TOK
        base_iota = lax.iota(jnp.int32, 16)
        cols = [jnp.full((16,), d, jnp.int32) for d in range(_DIM)]

        def start_gather(s, bi):
            pltpu.sync_copy(
                xt_hbm.at[pl.ds(s * _B + b0, _TOK)], idx_v.at[bi]
            )
            pltpu.async_copy(
                table_hbm.at[idx_v.at[bi]], rows_v.at[bi], gsem.at[bi]
            )

        def wait_gather(bi):
            pltpu.make_async_copy(
                table_hbm.at[idx_v.at[bi]], rows_v.at[bi], gsem.at[bi]
            ).wait()

        def start_write(s, bi):
            pltpu.async_copy(
                out_v.at[bi],
                out_hbm.at[s, :, pl.ds(_BT * wid, _BT)],
                osem.at[bi],
            )

        def wait_write(s, bi):
            pltpu.make_async_copy(
                out_v.at[bi],
                out_hbm.at[s, :, pl.ds(_BT * wid, _BT)],
                osem.at[bi],
            ).wait()

        def transpose(bi):
            def tbody(g, _):
                row_idx = base_iota + g * 16
                btl = lax.div(g, 8)
                lane = lax.rem(g, 8) * 16
                for d in range(_DIM):
                    v = plsc.load_gather(
                        rows_v.at[bi], [row_idx, cols[d]]
                    )
                    out_v[bi, d // 8, btl, d % 8, pl.ds(lane, 16)] = v
                return 0

            lax.fori_loop(0, _TOK // 16, tbody, 0)

        start_gather(0, 0)

        def body(i2, _):
            s0 = i2 * 2
            start_gather(s0 + 1, 1)
            wait_gather(0)

            @pl.when(i2 >= 1)
            def _():
                wait_write(s0 - 2, 0)

            transpose(0)
            start_write(s0, 0)

            @pl.when(i2 <= (_S // 2 - 2))
            def _():
                start_gather(s0 + 2, 0)

            wait_gather(1)

            @pl.when(i2 >= 1)
            def _():
                wait_write(s0 - 1, 1)

            transpose(1)
            start_write(s0 + 1, 1)
            return 0

        lax.fori_loop(0, _S // 2, body, 0)
        wait_write(_S - 2, 0)
        wait_write(_S - 1, 1)

    return k(xt_flat, table)


def kernel(x, table):
    xt_flat = x.T.reshape(_S * _B).astype(jnp.int32)
    out5 = _embed_gather_t(xt_flat, table)
    return out5.transpose(2, 4, 0, 1, 3).reshape(_B, _S, _DIM)
